# Initial kernel scaffold; baseline (speedup 1.0000x reference)
import jax
import jax.numpy as jnp
from jax.experimental import pallas as pl


def kernel(z, view_ids):
    raise NotImplementedError("write your pallas kernel here")


# speedup vs baseline: 1.3318x; 1.3318x over previous
import functools

import jax
import jax.numpy as jnp
from jax.experimental import pallas as pl
from jax.experimental.pallas import tpu as pltpu

_GAMMA = 20
_LAMBDA_INV = 25.0
_NEG_BIG = -3.4e38


def _stage1_body(vidr_ref, vidc_ref, z_ref, best_ref, bj_ref, mdot_ref,
                 norm_ref, *, bb_per_prog, P):
    vidr = vidr_ref[...]
    vidc = vidc_ref[...]
    row_ids = jax.lax.broadcasted_iota(jnp.int32, (P, P), 0)
    col_ids = jax.lax.broadcasted_iota(jnp.int32, (P, P), 1)
    diag = row_ids == col_ids
    mask = (vidc != vidr) & jnp.logical_not(diag)
    base = pl.program_id(0) * bb_per_prog
    for bb in range(bb_per_prog):
        zb = z_ref[bb]
        sim = jax.lax.dot_general(
            zb, zb, (((1,), (1,)), ((), ())),
            preferred_element_type=jnp.float32)
        masked = jnp.where(mask, sim, -1.0)
        best = jnp.max(masked, axis=0, keepdims=True)
        is_max = masked == best
        j = jnp.min(jnp.where(is_max, row_ids, P), axis=0, keepdims=True)
        mdot = jnp.max(jnp.where(row_ids == j, sim, _NEG_BIG),
                       axis=0, keepdims=True)
        norm = jnp.max(jnp.where(diag, sim, _NEG_BIG),
                       axis=0, keepdims=True)
        best_ref[pl.ds(bb, 1), :] = best
        bj_ref[pl.ds(bb, 1), :] = j + (base + bb) * P
        mdot_ref[pl.ds(bb, 1), :] = mdot
        norm_ref[pl.ds(bb, 1), :] = norm


def _stage2_body(best_ref, bj_ref, mdot_ref, norm_ref, loss_ref, cos_ref,
                 *, B, P, D):
    s = best_ref[...]
    bj = bj_ref[...]
    md = mdot_ref[...]
    nm = norm_ref[...]
    flat = (jax.lax.broadcasted_iota(jnp.int32, (B, P), 0) * P
            + jax.lax.broadcasted_iota(jnp.int32, (B, P), 1))
    vsum = jnp.zeros((1, 1), jnp.float32)
    lsum = jnp.zeros((1, 1), jnp.float32)
    for _ in range(_GAMMA):
        v = jnp.max(s, keepdims=True)
        idx = jnp.min(jnp.where(s == v, flat, B * P),
                      keepdims=True)
        sel = flat == idx
        n1 = jnp.sum(jnp.where(sel, nm, 0.0), keepdims=True)
        d = jnp.sum(jnp.where(sel, md, 0.0), keepdims=True)
        j = jnp.sum(jnp.where(sel, bj, 0), keepdims=True)
        n2 = jnp.sum(jnp.where(flat == j, nm, 0.0), keepdims=True)
        s = jnp.where(sel, _NEG_BIG, s)
        vsum = vsum + v
        lsum = lsum + (n1 + n2 - 2.0 * d)
    loss_ref[...] = (_LAMBDA_INV / (_GAMMA * D)) * lsum
    cos_ref[...] = vsum / _GAMMA


def kernel(z, view_ids):
    B, P, D = z.shape
    BB = 8
    vid = view_ids.astype(jnp.int32)
    vidr = vid.reshape(1, P)
    vidc = vid.reshape(P, 1)
    f32 = jnp.float32
    best, bj, mdot, norm = pl.pallas_call(
        functools.partial(_stage1_body, bb_per_prog=BB, P=P),
        grid=(B // BB,),
        in_specs=[
            pl.BlockSpec((1, P), lambda i: (0, 0)),
            pl.BlockSpec((P, 1), lambda i: (0, 0)),
            pl.BlockSpec((BB, P, D), lambda i: (i, 0, 0)),
        ],
        out_specs=[
            pl.BlockSpec((BB, P), lambda i: (i, 0)),
            pl.BlockSpec((BB, P), lambda i: (i, 0)),
            pl.BlockSpec((BB, P), lambda i: (i, 0)),
            pl.BlockSpec((BB, P), lambda i: (i, 0)),
        ],
        out_shape=[
            jax.ShapeDtypeStruct((B, P), f32),
            jax.ShapeDtypeStruct((B, P), jnp.int32),
            jax.ShapeDtypeStruct((B, P), f32),
            jax.ShapeDtypeStruct((B, P), f32),
        ],
        compiler_params=pltpu.CompilerParams(
            dimension_semantics=("parallel",),
            vmem_limit_bytes=48 * 1024 * 1024,
        ),
        name="fmatch_sim_stage1",
    )(vidr, vidc, z)
    loss2, cos2 = pl.pallas_call(
        functools.partial(_stage2_body, B=B, P=P, D=D),
        out_shape=[
            jax.ShapeDtypeStruct((1, 1), f32),
            jax.ShapeDtypeStruct((1, 1), f32),
        ],
        name="fmatch_sim_stage2",
    )(best, bj, mdot, norm)
    return loss2[0, 0], cos2[0, 0]


# speedup vs baseline: 1.6547x; 1.2425x over previous
import functools

import jax
import jax.numpy as jnp
from jax.experimental import pallas as pl
from jax.experimental.pallas import tpu as pltpu

_GAMMA = 20
_LAMBDA_INV = 25.0
_NEG_BIG = -3.4e38
_BIG_I = 2 ** 30


def _fold_rows(v, idx, frides, irides):
    while v.shape[0] > 8:
        h = v.shape[0] // 2
        take = v[:h] >= v[h:]
        v = jnp.where(take, v[:h], v[h:])
        idx = jnp.where(take, idx[:h], idx[h:])
        frides = [jnp.where(take, r[:h], r[h:]) for r in frides]
        irides = [jnp.where(take, r[:h], r[h:]) for r in irides]
    vw = jnp.max(v, axis=0, keepdims=True)
    iw = jnp.min(jnp.where(v == vw, idx, _BIG_I), axis=0, keepdims=True)
    sel = idx == iw
    fr = [jnp.max(jnp.where(sel, r, _NEG_BIG), axis=0, keepdims=True)
          for r in frides]
    ir = [jnp.max(jnp.where(sel, r, -1), axis=0, keepdims=True)
          for r in irides]
    return vw, iw, fr, ir


def _stage1_body(vidr_ref, vidc_ref, z_ref, best_ref, bj_ref, mdot_ref,
                 norm_ref, *, bb_per_prog, P):
    vidr = vidr_ref[...]
    vidc = vidc_ref[...]
    row_ids = jax.lax.broadcasted_iota(jnp.int32, (P, P), 0)
    col_ids = jax.lax.broadcasted_iota(jnp.int32, (P, P), 1)
    mask = (vidc != vidr) & (row_ids != col_ids)
    diag128 = (jax.lax.broadcasted_iota(jnp.int32, (128, 128), 0)
               == jax.lax.broadcasted_iota(jnp.int32, (128, 128), 1))
    base = pl.program_id(0) * bb_per_prog
    for bb in range(bb_per_prog):
        zb = z_ref[bb]
        sim = jax.lax.dot_general(
            zb, zb, (((1,), (1,)), ((), ())),
            preferred_element_type=jnp.float32)
        masked = jnp.where(mask, sim, -1.0)
        best, j, (mdot,), _ = _fold_rows(masked, row_ids, [sim], [])
        norm = jnp.concatenate(
            [jnp.max(jnp.where(diag128,
                               sim[t * 128:(t + 1) * 128,
                                   t * 128:(t + 1) * 128], _NEG_BIG),
                     axis=0, keepdims=True)
             for t in range(P // 128)], axis=1)
        best_ref[pl.ds(bb, 1), :] = best
        bj_ref[pl.ds(bb, 1), :] = j + (base + bb) * P
        mdot_ref[pl.ds(bb, 1), :] = mdot
        norm_ref[pl.ds(bb, 1), :] = norm


def _stage2_body(best_ref, bj_ref, mdot_ref, norm_ref, loss_ref, cos_ref,
                 *, B, P, D):
    s = best_ref[...]
    bj = bj_ref[...]
    md = mdot_ref[...]
    nm = norm_ref[...]
    flat = (jax.lax.broadcasted_iota(jnp.int32, (B, P), 0) * P
            + jax.lax.broadcasted_iota(jnp.int32, (B, P), 1))
    vsum = jnp.zeros((1, 1), jnp.float32)
    lsum = jnp.zeros((1, 1), jnp.float32)
    cnt = jnp.zeros((B, P), jnp.float32)
    for _ in range(_GAMMA):
        vw, iw, (nmw, mdw), (bjw,) = _fold_rows(s, flat, [nm, md], [bj])
        v1 = jnp.max(vw, axis=1, keepdims=True)
        iw1 = jnp.min(jnp.where(vw == v1, iw, _BIG_I), axis=1, keepdims=True)
        sel1 = iw == iw1
        n1 = jnp.max(jnp.where(sel1, nmw, _NEG_BIG), axis=1, keepdims=True)
        d = jnp.max(jnp.where(sel1, mdw, _NEG_BIG), axis=1, keepdims=True)
        bj1 = jnp.max(jnp.where(sel1, bjw, -1), axis=1, keepdims=True)
        s = jnp.where(flat == iw1, _NEG_BIG, s)
        cnt = cnt + jnp.where(flat == bj1, 1.0, 0.0)
        vsum = vsum + v1
        lsum = lsum + (n1 - 2.0 * d)
    lsum = lsum + jnp.sum(nm * cnt, keepdims=True)
    loss_ref[...] = (_LAMBDA_INV / (_GAMMA * D)) * lsum
    cos_ref[...] = vsum / _GAMMA


def kernel(z, view_ids):
    B, P, D = z.shape
    BB = 8
    vid = view_ids.astype(jnp.int32)
    vidr = vid.reshape(1, P)
    vidc = vid.reshape(P, 1)
    f32 = jnp.float32
    best, bj, mdot, norm = pl.pallas_call(
        functools.partial(_stage1_body, bb_per_prog=BB, P=P),
        grid=(B // BB,),
        in_specs=[
            pl.BlockSpec((1, P), lambda i: (0, 0)),
            pl.BlockSpec((P, 1), lambda i: (0, 0)),
            pl.BlockSpec((BB, P, D), lambda i: (i, 0, 0)),
        ],
        out_specs=[
            pl.BlockSpec((BB, P), lambda i: (i, 0)),
            pl.BlockSpec((BB, P), lambda i: (i, 0)),
            pl.BlockSpec((BB, P), lambda i: (i, 0)),
            pl.BlockSpec((BB, P), lambda i: (i, 0)),
        ],
        out_shape=[
            jax.ShapeDtypeStruct((B, P), f32),
            jax.ShapeDtypeStruct((B, P), jnp.int32),
            jax.ShapeDtypeStruct((B, P), f32),
            jax.ShapeDtypeStruct((B, P), f32),
        ],
        compiler_params=pltpu.CompilerParams(
            dimension_semantics=("parallel",),
            vmem_limit_bytes=48 * 1024 * 1024,
        ),
        name="fmatch_sim_stage1",
    )(vidr, vidc, z)
    loss2, cos2 = pl.pallas_call(
        functools.partial(_stage2_body, B=B, P=P, D=D),
        out_shape=[
            jax.ShapeDtypeStruct((1, 1), f32),
            jax.ShapeDtypeStruct((1, 1), f32),
        ],
        name="fmatch_sim_stage2",
    )(best, bj, mdot, norm)
    return loss2[0, 0], cos2[0, 0]


# speedup vs baseline: 1.9971x; 1.2069x over previous
import functools

import jax
import jax.numpy as jnp
from jax.experimental import pallas as pl
from jax.experimental.pallas import tpu as pltpu

_GAMMA = 20
_LAMBDA_INV = 25.0
_NEG_BIG = -3.4e38
_BIG_I = 2 ** 30


def _fold_rows(v, idx, frides, irides):
    while v.shape[0] > 8:
        h = v.shape[0] // 2
        take = v[:h] >= v[h:]
        v = jnp.where(take, v[:h], v[h:])
        idx = jnp.where(take, idx[:h], idx[h:])
        frides = [jnp.where(take, r[:h], r[h:]) for r in frides]
        irides = [jnp.where(take, r[:h], r[h:]) for r in irides]
    vw = jnp.max(v, axis=0, keepdims=True)
    iw = jnp.min(jnp.where(v == vw, idx, _BIG_I), axis=0, keepdims=True)
    sel = idx == iw
    fr = [jnp.max(jnp.where(sel, r, _NEG_BIG), axis=0, keepdims=True)
          for r in frides]
    ir = [jnp.max(jnp.where(sel, r, -1), axis=0, keepdims=True)
          for r in irides]
    return vw, iw, fr, ir


def _stage1_body(vidr_ref, vidc_ref, z_ref, best_ref, bj_ref, norm_ref,
                 *, bb_per_prog, P):
    vidr = vidr_ref[...]
    vidc = vidc_ref[...]
    row_ids = jax.lax.broadcasted_iota(jnp.int32, (P, P), 0)
    col_ids = jax.lax.broadcasted_iota(jnp.int32, (P, P), 1)
    mask = (vidc != vidr) & (row_ids != col_ids)
    bias = jnp.where(mask, 0.0, _NEG_BIG)
    diag128 = (jax.lax.broadcasted_iota(jnp.int32, (128, 128), 0)
               == jax.lax.broadcasted_iota(jnp.int32, (128, 128), 1))
    base = pl.program_id(0) * bb_per_prog
    for bb in range(bb_per_prog):
        zb = z_ref[bb]
        sim = jax.lax.dot_general(
            zb, zb, (((1,), (1,)), ((), ())),
            preferred_element_type=jnp.float32)
        vraw, j, _, _ = _fold_rows(sim + bias, row_ids, [], [])
        best = jnp.maximum(vraw, -1.0)
        norm = jnp.concatenate(
            [jnp.max(jnp.where(diag128,
                               sim[t * 128:(t + 1) * 128,
                                   t * 128:(t + 1) * 128], _NEG_BIG),
                     axis=0, keepdims=True)
             for t in range(P // 128)], axis=1)
        best_ref[pl.ds(bb, 1), :] = best
        bj_ref[pl.ds(bb, 1), :] = j + (base + bb) * P
        norm_ref[pl.ds(bb, 1), :] = norm


def _stage2_body(best_ref, bj_ref, norm_ref, loss_ref, cos_ref, *, B, P, D):
    s = best_ref[...]
    bj = bj_ref[...]
    nm = norm_ref[...]
    flat = (jax.lax.broadcasted_iota(jnp.int32, (B, P), 0) * P
            + jax.lax.broadcasted_iota(jnp.int32, (B, P), 1))
    vsum = jnp.zeros((1, 1), jnp.float32)
    lsum = jnp.zeros((1, 1), jnp.float32)
    cnt = jnp.zeros((B, P), jnp.float32)
    for _ in range(_GAMMA):
        vw, iw, (nmw,), (bjw,) = _fold_rows(s, flat, [nm], [bj])
        v1 = jnp.max(vw, axis=1, keepdims=True)
        iw1 = jnp.min(jnp.where(vw == v1, iw, _BIG_I), axis=1, keepdims=True)
        sel1 = iw == iw1
        n1 = jnp.max(jnp.where(sel1, nmw, _NEG_BIG), axis=1, keepdims=True)
        bj1 = jnp.max(jnp.where(sel1, bjw, -1), axis=1, keepdims=True)
        s = jnp.where(flat == iw1, _NEG_BIG, s)
        cnt = cnt + jnp.where(flat == bj1, 1.0, 0.0)
        vsum = vsum + v1
        lsum = lsum + (n1 - 2.0 * v1)
    lsum = lsum + jnp.sum(nm * cnt, keepdims=True)
    loss_ref[...] = (_LAMBDA_INV / (_GAMMA * D)) * lsum
    cos_ref[...] = vsum / _GAMMA


def kernel(z, view_ids):
    B, P, D = z.shape
    BB = 8
    vid = view_ids.astype(jnp.int32)
    vidr = vid.reshape(1, P)
    vidc = vid.reshape(P, 1)
    f32 = jnp.float32
    best, bj, norm = pl.pallas_call(
        functools.partial(_stage1_body, bb_per_prog=BB, P=P),
        grid=(B // BB,),
        in_specs=[
            pl.BlockSpec((1, P), lambda i: (0, 0)),
            pl.BlockSpec((P, 1), lambda i: (0, 0)),
            pl.BlockSpec((BB, P, D), lambda i: (i, 0, 0)),
        ],
        out_specs=[
            pl.BlockSpec((BB, P), lambda i: (i, 0)),
            pl.BlockSpec((BB, P), lambda i: (i, 0)),
            pl.BlockSpec((BB, P), lambda i: (i, 0)),
        ],
        out_shape=[
            jax.ShapeDtypeStruct((B, P), f32),
            jax.ShapeDtypeStruct((B, P), jnp.int32),
            jax.ShapeDtypeStruct((B, P), f32),
        ],
        compiler_params=pltpu.CompilerParams(
            dimension_semantics=("parallel",),
            vmem_limit_bytes=48 * 1024 * 1024,
        ),
        name="fmatch_sim_stage1",
    )(vidr, vidc, z)
    loss2, cos2 = pl.pallas_call(
        functools.partial(_stage2_body, B=B, P=P, D=D),
        out_shape=[
            jax.ShapeDtypeStruct((1, 1), f32),
            jax.ShapeDtypeStruct((1, 1), f32),
        ],
        name="fmatch_sim_stage2",
    )(best, bj, norm)
    return loss2[0, 0], cos2[0, 0]


# speedup vs baseline: 2.0315x; 1.0172x over previous
import functools

import jax
import jax.numpy as jnp
from jax.experimental import pallas as pl
from jax.experimental.pallas import tpu as pltpu

_GAMMA = 20
_LAMBDA_INV = 25.0
_NEG_BIG = -3.4e38
_BIG_I = 2 ** 30


def _fold_rows(v, idx, irides):
    while v.shape[0] > 8:
        h = v.shape[0] // 2
        take = v[:h] >= v[h:]
        v = jnp.where(take, v[:h], v[h:])
        idx = jnp.where(take, idx[:h], idx[h:])
        irides = [jnp.where(take, r[:h], r[h:]) for r in irides]
    vw = jnp.max(v, axis=0, keepdims=True)
    iw = jnp.min(jnp.where(v == vw, idx, _BIG_I), axis=0, keepdims=True)
    sel = idx == iw
    ir = [jnp.max(jnp.where(sel, r, -1), axis=0, keepdims=True)
          for r in irides]
    return vw, iw, ir


def _body(vidr_ref, vidc_ref, z_ref, loss_ref, cos_ref,
          sbest_ref, sbj_ref, snorm_ref, bias_ref, *, bb_per_prog, nprog,
          B, P, D):
    i = pl.program_id(0)

    @pl.when(i == 0)
    def _():
        vidr = vidr_ref[...]
        vidc = vidc_ref[...]
        rids = jax.lax.broadcasted_iota(jnp.int32, (P, P), 0)
        cids = jax.lax.broadcasted_iota(jnp.int32, (P, P), 1)
        mask = (vidc != vidr) & (rids != cids)
        bias_ref[...] = jnp.where(mask, 0.0, _NEG_BIG)

    bias = bias_ref[...]
    row_ids = jax.lax.broadcasted_iota(jnp.int32, (P, P), 0)
    diag128 = (jax.lax.broadcasted_iota(jnp.int32, (128, 128), 0)
               == jax.lax.broadcasted_iota(jnp.int32, (128, 128), 1))
    base = i * bb_per_prog
    for bb in range(bb_per_prog):
        zb = z_ref[bb]
        sim = jax.lax.dot_general(
            zb, zb, (((1,), (1,)), ((), ())),
            preferred_element_type=jnp.float32)
        vraw, j, _ = _fold_rows(sim + bias, row_ids, [])
        best = jnp.maximum(vraw, -1.0)
        norm = jnp.concatenate(
            [jnp.max(jnp.where(diag128,
                               sim[t * 128:(t + 1) * 128,
                                   t * 128:(t + 1) * 128], _NEG_BIG),
                     axis=0, keepdims=True)
             for t in range(P // 128)], axis=1)
        sbest_ref[pl.ds(i, 1), pl.ds(bb, 1), :] = best[None]
        sbj_ref[pl.ds(i, 1), pl.ds(bb, 1), :] = (j + (base + bb) * P)[None]
        snorm_ref[pl.ds(i, 1), pl.ds(bb, 1), :] = norm[None]

    @pl.when(i == nprog - 1)
    def _():
        s = sbest_ref[...].reshape(B, P)
        bjg = sbj_ref[...].reshape(B, P)
        nm = snorm_ref[...].reshape(B, P)
        flat = (jax.lax.broadcasted_iota(jnp.int32, (B, P), 0) * P
                + jax.lax.broadcasted_iota(jnp.int32, (B, P), 1))
        lane = jax.lax.broadcasted_iota(jnp.int32, (1, P), 1)
        vsum = jnp.zeros((1, 1), jnp.float32)
        cnt = jnp.zeros((B, P), jnp.float32)
        for _k in range(_GAMMA):
            vw, iw, (bjw,) = _fold_rows(s, flat, [bjg])
            v1 = jnp.max(vw, axis=1, keepdims=True)
            i1 = jnp.argmax(vw, axis=1, keepdims=True)
            sel1 = lane == i1
            iw1 = jnp.max(jnp.where(sel1, iw, -1), axis=1, keepdims=True)
            bj1 = jnp.max(jnp.where(sel1, bjw, -1), axis=1, keepdims=True)
            win2d = flat == iw1
            s = jnp.where(win2d, _NEG_BIG, s)
            cnt = (cnt + jnp.where(win2d, 1.0, 0.0)
                   + jnp.where(flat == bj1, 1.0, 0.0))
            vsum = vsum + v1
        nsum = jnp.sum(nm * cnt, keepdims=True)
        loss_ref[...] = (_LAMBDA_INV / (_GAMMA * D)) * (nsum - 2.0 * vsum)
        cos_ref[...] = vsum / _GAMMA


def kernel(z, view_ids):
    B, P, D = z.shape
    BB = 8
    nprog = B // BB
    vid = view_ids.astype(jnp.int32)
    vidr = vid.reshape(1, P)
    vidc = vid.reshape(P, 1)
    f32 = jnp.float32
    loss2, cos2 = pl.pallas_call(
        functools.partial(_body, bb_per_prog=BB, nprog=nprog, B=B, P=P, D=D),
        grid=(nprog,),
        in_specs=[
            pl.BlockSpec((1, P), lambda i: (0, 0)),
            pl.BlockSpec((P, 1), lambda i: (0, 0)),
            pl.BlockSpec((BB, P, D), lambda i: (i, 0, 0)),
        ],
        out_specs=[
            pl.BlockSpec((1, 1), lambda i: (0, 0)),
            pl.BlockSpec((1, 1), lambda i: (0, 0)),
        ],
        out_shape=[
            jax.ShapeDtypeStruct((1, 1), f32),
            jax.ShapeDtypeStruct((1, 1), f32),
        ],
        scratch_shapes=[
            pltpu.VMEM((nprog, BB, P), f32),
            pltpu.VMEM((nprog, BB, P), jnp.int32),
            pltpu.VMEM((nprog, BB, P), f32),
            pltpu.VMEM((P, P), f32),
        ],
        compiler_params=pltpu.CompilerParams(
            dimension_semantics=("arbitrary",),
            vmem_limit_bytes=48 * 1024 * 1024,
        ),
        name="fmatch_sim_fused",
    )(vidr, vidc, z)
    return loss2[0, 0], cos2[0, 0]


# speedup vs baseline: 2.1459x; 1.0563x over previous
import functools

import jax
import jax.numpy as jnp
from jax.experimental import pallas as pl
from jax.experimental.pallas import tpu as pltpu

_GAMMA = 20
_LAMBDA_INV = 25.0
_NEG_BIG = -3.4e38
_BIG_I = 2 ** 30


def _fold_rows(v, idx, irides):
    while v.shape[0] > 8:
        h = v.shape[0] // 2
        take = v[:h] >= v[h:]
        v = jnp.where(take, v[:h], v[h:])
        idx = jnp.where(take, idx[:h], idx[h:])
        irides = [jnp.where(take, r[:h], r[h:]) for r in irides]
    vw = jnp.max(v, axis=0, keepdims=True)
    iw = jnp.min(jnp.where(v == vw, idx, _BIG_I), axis=0, keepdims=True)
    sel = idx == iw
    ir = [jnp.max(jnp.where(sel, r, -1), axis=0, keepdims=True)
          for r in irides]
    return vw, iw, ir


def _body(vidr_ref, vidc_ref, z_ref, loss_ref, cos_ref,
          sbest_ref, sbj_ref, snorm_ref, bias_ref, *, bb_per_prog, nprog,
          B, P, D):
    i = pl.program_id(0)

    @pl.when(i == 0)
    def _():
        vidr = vidr_ref[...]
        vidc = vidc_ref[...]
        rids = jax.lax.broadcasted_iota(jnp.int32, (P, P), 0)
        cids = jax.lax.broadcasted_iota(jnp.int32, (P, P), 1)
        mask = (vidc != vidr) & (rids != cids)
        bias_ref[...] = jnp.where(mask, 0.0, _NEG_BIG)

    bias = bias_ref[...]
    row_ids = jax.lax.broadcasted_iota(jnp.int32, (P, P), 0)
    diag128 = (jax.lax.broadcasted_iota(jnp.int32, (128, 128), 0)
               == jax.lax.broadcasted_iota(jnp.int32, (128, 128), 1))
    base = i * bb_per_prog
    for bb in range(bb_per_prog):
        zb = z_ref[bb]
        sim = jax.lax.dot_general(
            zb, zb, (((1,), (1,)), ((), ())),
            preferred_element_type=jnp.float32)
        vraw, j, _ = _fold_rows(sim + bias, row_ids, [])
        best = jnp.maximum(vraw, -1.0)
        norm = jnp.concatenate(
            [jnp.max(jnp.where(diag128,
                               sim[t * 128:(t + 1) * 128,
                                   t * 128:(t + 1) * 128], _NEG_BIG),
                     axis=0, keepdims=True)
             for t in range(P // 128)], axis=1)
        sbest_ref[pl.ds(i, 1), pl.ds(bb, 1), :] = best[None]
        sbj_ref[pl.ds(i, 1), pl.ds(bb, 1), :] = (j + (base + bb) * P)[None]
        snorm_ref[pl.ds(i, 1), pl.ds(bb, 1), :] = norm[None]

    @pl.when(i == nprog - 1)
    def _():
        s = sbest_ref[...].reshape(B, P)
        nm = snorm_ref[...].reshape(B, P)
        bjf = sbj_ref[...].reshape(B, P).astype(jnp.float32)
        flat = (jax.lax.broadcasted_iota(jnp.int32, (B, P), 0) * P
                + jax.lax.broadcasted_iota(jnp.int32, (B, P), 1))
        lane = jax.lax.broadcasted_iota(jnp.int32, (1, P), 1)
        vsum = jnp.zeros((1, 1), jnp.float32)
        cnt = jnp.zeros((B, P), jnp.float32)
        for _k in range(_GAMMA):
            vw = jnp.max(s, axis=0, keepdims=True)
            v1 = jnp.max(vw, axis=1, keepdims=True)
            i1 = jnp.argmax(vw, axis=1, keepdims=True)
            colf = jnp.where(lane == i1, 1.0, 0.0)
            wmask = jnp.where(s == v1, colf, 0.0)
            s = s + wmask * _NEG_BIG
            cnt = cnt + wmask
            bj1 = jnp.sum(bjf * wmask, keepdims=True).astype(jnp.int32)
            cnt = cnt + jnp.where(flat == bj1, 1.0, 0.0)
            vsum = vsum + v1
        nsum = jnp.sum(nm * cnt, keepdims=True)
        loss_ref[...] = (_LAMBDA_INV / (_GAMMA * D)) * (nsum - 2.0 * vsum)
        cos_ref[...] = vsum / _GAMMA


def kernel(z, view_ids):
    B, P, D = z.shape
    BB = 8
    nprog = B // BB
    vid = view_ids.astype(jnp.int32)
    vidr = vid.reshape(1, P)
    vidc = vid.reshape(P, 1)
    f32 = jnp.float32
    loss2, cos2 = pl.pallas_call(
        functools.partial(_body, bb_per_prog=BB, nprog=nprog, B=B, P=P, D=D),
        grid=(nprog,),
        in_specs=[
            pl.BlockSpec((1, P), lambda i: (0, 0)),
            pl.BlockSpec((P, 1), lambda i: (0, 0)),
            pl.BlockSpec((BB, P, D), lambda i: (i, 0, 0)),
        ],
        out_specs=[
            pl.BlockSpec((1, 1), lambda i: (0, 0)),
            pl.BlockSpec((1, 1), lambda i: (0, 0)),
        ],
        out_shape=[
            jax.ShapeDtypeStruct((1, 1), f32),
            jax.ShapeDtypeStruct((1, 1), f32),
        ],
        scratch_shapes=[
            pltpu.VMEM((nprog, BB, P), f32),
            pltpu.VMEM((nprog, BB, P), jnp.int32),
            pltpu.VMEM((nprog, BB, P), f32),
            pltpu.VMEM((P, P), f32),
        ],
        compiler_params=pltpu.CompilerParams(
            dimension_semantics=("arbitrary",),
            vmem_limit_bytes=48 * 1024 * 1024,
        ),
        name="fmatch_sim_fused",
    )(vidr, vidc, z)
    return loss2[0, 0], cos2[0, 0]


# speedup vs baseline: 2.1514x; 1.0026x over previous
import functools

import jax
import jax.numpy as jnp
from jax.experimental import pallas as pl
from jax.experimental.pallas import tpu as pltpu

_GAMMA = 20
_LAMBDA_INV = 25.0
_NEG_BIG = -3.4e38
_BIG_I = 2 ** 30


def _fold_rows(v, idx, irides):
    while v.shape[0] > 8:
        h = v.shape[0] // 2
        take = v[:h] >= v[h:]
        v = jnp.where(take, v[:h], v[h:])
        idx = jnp.where(take, idx[:h], idx[h:])
        irides = [jnp.where(take, r[:h], r[h:]) for r in irides]
    vw = jnp.max(v, axis=0, keepdims=True)
    iw = jnp.min(jnp.where(v == vw, idx, _BIG_I), axis=0, keepdims=True)
    sel = idx == iw
    ir = [jnp.max(jnp.where(sel, r, -1), axis=0, keepdims=True)
          for r in irides]
    return vw, iw, ir


def _body(vidr_ref, vidc_ref, z_ref, loss_ref, cos_ref,
          sbest_ref, sbj_ref, snorm_ref, bias_ref, *, bb_per_prog, nprog,
          B, P, D):
    i = pl.program_id(0)

    @pl.when(i == 0)
    def _():
        vidr = vidr_ref[...]
        vidc = vidc_ref[...]
        rids = jax.lax.broadcasted_iota(jnp.int32, (P, P), 0)
        cids = jax.lax.broadcasted_iota(jnp.int32, (P, P), 1)
        mask = (vidc != vidr) & (rids != cids)
        bias_ref[...] = jnp.where(mask, 0.0, _NEG_BIG)

    bias = bias_ref[...]
    row_ids = jax.lax.broadcasted_iota(jnp.int32, (P, P), 0)
    diag128 = (jax.lax.broadcasted_iota(jnp.int32, (128, 128), 0)
               == jax.lax.broadcasted_iota(jnp.int32, (128, 128), 1))
    base = i * bb_per_prog
    for bb in range(bb_per_prog):
        zb = z_ref[bb]
        sim = jax.lax.dot_general(
            zb, zb, (((1,), (1,)), ((), ())),
            preferred_element_type=jnp.float32)
        vraw, j, _ = _fold_rows(sim + bias, row_ids, [])
        best = jnp.maximum(vraw, -1.0)
        norm = jnp.concatenate(
            [jnp.max(jnp.where(diag128,
                               sim[t * 128:(t + 1) * 128,
                                   t * 128:(t + 1) * 128], _NEG_BIG),
                     axis=0, keepdims=True)
             for t in range(P // 128)], axis=1)
        sbest_ref[pl.ds(i, 1), pl.ds(bb, 1), :] = best[None]
        sbj_ref[pl.ds(i, 1), pl.ds(bb, 1), :] = (
            (j + (base + bb) * P).astype(jnp.float32)[None])
        snorm_ref[pl.ds(i, 1), pl.ds(bb, 1), :] = norm[None]

    @pl.when(i == nprog - 1)
    def _():
        s = sbest_ref[...].reshape(B, P)
        nm = snorm_ref[...].reshape(B, P)
        bjf = sbj_ref[...].reshape(B, P)
        flatf = (jax.lax.broadcasted_iota(jnp.int32, (B, P), 0) * P
                 + jax.lax.broadcasted_iota(jnp.int32, (B, P), 1)
                 ).astype(jnp.float32)
        lane = jax.lax.broadcasted_iota(jnp.int32, (1, P), 1)
        vsum = jnp.zeros((1, 1), jnp.float32)
        cnt = jnp.zeros((B, P), jnp.float32)
        for _k in range(_GAMMA):
            vw = jnp.max(s, axis=0, keepdims=True)
            v1 = jnp.max(vw, axis=1, keepdims=True)
            i1 = jnp.argmax(vw, axis=1, keepdims=True)
            colf = jnp.where(lane == i1, 1.0, 0.0)
            wmask = jnp.where(s == v1, colf, 0.0)
            s = s + wmask * _NEG_BIG
            cnt = cnt + wmask
            bj1 = jnp.sum(bjf * wmask, keepdims=True)
            cnt = cnt + jnp.where(flatf == bj1, 1.0, 0.0)
            vsum = vsum + v1
        nsum = jnp.sum(nm * cnt, keepdims=True)
        loss_ref[...] = (_LAMBDA_INV / (_GAMMA * D)) * (nsum - 2.0 * vsum)
        cos_ref[...] = vsum / _GAMMA


def kernel(z, view_ids):
    B, P, D = z.shape
    BB = 8
    nprog = B // BB
    vid = view_ids.astype(jnp.int32)
    vidr = vid.reshape(1, P)
    vidc = vid.reshape(P, 1)
    f32 = jnp.float32
    loss2, cos2 = pl.pallas_call(
        functools.partial(_body, bb_per_prog=BB, nprog=nprog, B=B, P=P, D=D),
        grid=(nprog,),
        in_specs=[
            pl.BlockSpec((1, P), lambda i: (0, 0)),
            pl.BlockSpec((P, 1), lambda i: (0, 0)),
            pl.BlockSpec((BB, P, D), lambda i: (i, 0, 0)),
        ],
        out_specs=[
            pl.BlockSpec((1, 1), lambda i: (0, 0)),
            pl.BlockSpec((1, 1), lambda i: (0, 0)),
        ],
        out_shape=[
            jax.ShapeDtypeStruct((1, 1), f32),
            jax.ShapeDtypeStruct((1, 1), f32),
        ],
        scratch_shapes=[
            pltpu.VMEM((nprog, BB, P), f32),
            pltpu.VMEM((nprog, BB, P), f32),
            pltpu.VMEM((nprog, BB, P), f32),
            pltpu.VMEM((P, P), f32),
        ],
        compiler_params=pltpu.CompilerParams(
            dimension_semantics=("arbitrary",),
            vmem_limit_bytes=48 * 1024 * 1024,
        ),
        name="fmatch_sim_fused",
    )(vidr, vidc, z)
    return loss2[0, 0], cos2[0, 0]


# speedup vs baseline: 2.1561x; 1.0022x over previous
import functools

import jax
import jax.numpy as jnp
from jax.experimental import pallas as pl
from jax.experimental.pallas import tpu as pltpu

_GAMMA = 20
_LAMBDA_INV = 25.0
_NEG_BIG = -3.4e38
_BIG_I = 2 ** 30


def _fold_rows(v, idx, irides):
    while v.shape[0] > 8:
        h = v.shape[0] // 2
        take = v[:h] >= v[h:]
        v = jnp.where(take, v[:h], v[h:])
        idx = jnp.where(take, idx[:h], idx[h:])
        irides = [jnp.where(take, r[:h], r[h:]) for r in irides]
    vw = jnp.max(v, axis=0, keepdims=True)
    iw = jnp.min(jnp.where(v == vw, idx, _BIG_I), axis=0, keepdims=True)
    sel = idx == iw
    ir = [jnp.max(jnp.where(sel, r, -1), axis=0, keepdims=True)
          for r in irides]
    return vw, iw, ir


def _body(vidr_ref, vidc_ref, z_ref, loss_ref, cos_ref,
          sbest_ref, sbj_ref, snorm_ref, bias_ref, *, bb_per_prog, nprog,
          B, P, D):
    i = pl.program_id(0)

    @pl.when(i == 0)
    def _():
        vidr = vidr_ref[...]
        vidc = vidc_ref[...]
        rids = jax.lax.broadcasted_iota(jnp.int32, (P, P), 0)
        cids = jax.lax.broadcasted_iota(jnp.int32, (P, P), 1)
        mask = (vidc != vidr) & (rids != cids)
        bias_ref[...] = jnp.where(mask, 0.0, _NEG_BIG)

    bias = bias_ref[...]
    row_ids = jax.lax.broadcasted_iota(jnp.int32, (P, P), 0)
    diag128 = (jax.lax.broadcasted_iota(jnp.int32, (128, 128), 0)
               == jax.lax.broadcasted_iota(jnp.int32, (128, 128), 1))
    base = i * bb_per_prog
    for bb in range(bb_per_prog):
        zb = z_ref[bb]
        sim = jax.lax.dot_general(
            zb, zb, (((1,), (1,)), ((), ())),
            preferred_element_type=jnp.float32)
        vraw, j, _ = _fold_rows(sim + bias, row_ids, [])
        best = jnp.maximum(vraw, -1.0)
        norm = jnp.concatenate(
            [jnp.max(jnp.where(diag128,
                               sim[t * 128:(t + 1) * 128,
                                   t * 128:(t + 1) * 128], _NEG_BIG),
                     axis=0, keepdims=True)
             for t in range(P // 128)], axis=1)
        sbest_ref[pl.ds(i, 1), pl.ds(bb, 1), :] = best[None]
        sbj_ref[pl.ds(i, 1), pl.ds(bb, 1), :] = (
            (j + (base + bb) * P).astype(jnp.float32)[None])
        snorm_ref[pl.ds(i, 1), pl.ds(bb, 1), :] = norm[None]

    @pl.when(i == nprog - 1)
    def _():
        s = sbest_ref[...].reshape(B, P)
        nm = snorm_ref[...].reshape(B, P)
        bjf = sbj_ref[...].reshape(B, P)
        flatf = (jax.lax.broadcasted_iota(jnp.int32, (B, P), 0) * P
                 + jax.lax.broadcasted_iota(jnp.int32, (B, P), 1)
                 ).astype(jnp.float32)
        lane = jax.lax.broadcasted_iota(jnp.int32, (1, P), 1)
        vsum = jnp.zeros((1, 1), jnp.float32)
        cnt = jnp.zeros((B, P), jnp.float32)
        for _k in range(_GAMMA):
            vw = jnp.max(s, axis=0, keepdims=True)
            v1 = jnp.max(vw, axis=1, keepdims=True)
            i1 = jnp.argmax(vw, axis=1, keepdims=True)
            colf = jnp.where(lane == i1, 1.0, 0.0)
            wmask = jnp.where(s == v1, colf, 0.0)
            s = s + wmask * _NEG_BIG
            cnt = cnt + wmask
            bj1 = jnp.sum(bjf * wmask, keepdims=True)
            cnt = cnt + jnp.where(flatf == bj1, 1.0, 0.0)
            vsum = vsum + v1
        nsum = jnp.sum(nm * cnt, keepdims=True)
        loss_ref[...] = (_LAMBDA_INV / (_GAMMA * D)) * (nsum - 2.0 * vsum)
        cos_ref[...] = vsum / _GAMMA


def kernel(z, view_ids):
    B, P, D = z.shape
    BB = 16
    nprog = B // BB
    vid = view_ids.astype(jnp.int32)
    vidr = vid.reshape(1, P)
    vidc = vid.reshape(P, 1)
    f32 = jnp.float32
    loss2, cos2 = pl.pallas_call(
        functools.partial(_body, bb_per_prog=BB, nprog=nprog, B=B, P=P, D=D),
        grid=(nprog,),
        in_specs=[
            pl.BlockSpec((1, P), lambda i: (0, 0)),
            pl.BlockSpec((P, 1), lambda i: (0, 0)),
            pl.BlockSpec((BB, P, D), lambda i: (i, 0, 0)),
        ],
        out_specs=[
            pl.BlockSpec((1, 1), lambda i: (0, 0)),
            pl.BlockSpec((1, 1), lambda i: (0, 0)),
        ],
        out_shape=[
            jax.ShapeDtypeStruct((1, 1), f32),
            jax.ShapeDtypeStruct((1, 1), f32),
        ],
        scratch_shapes=[
            pltpu.VMEM((nprog, BB, P), f32),
            pltpu.VMEM((nprog, BB, P), f32),
            pltpu.VMEM((nprog, BB, P), f32),
            pltpu.VMEM((P, P), f32),
        ],
        compiler_params=pltpu.CompilerParams(
            dimension_semantics=("arbitrary",),
            vmem_limit_bytes=48 * 1024 * 1024,
        ),
        name="fmatch_sim_fused",
    )(vidr, vidc, z)
    return loss2[0, 0], cos2[0, 0]
